# baseline (speedup 1.0000x reference)
import functools

import jax
import jax.numpy as jnp
from jax import lax
from jax.experimental import pallas as pl
from jax.experimental.pallas import tpu as pltpu
from jax.experimental.pallas import tpu_sc as plsc

S = 2048
D_MODEL = 768
D_FFN = 1536
N_EXPERTS = 8
TOP_K = 2

TBLK = 256
EPAD = 128
GBLK = 512
NA = S * TOP_K
NB = NA // GBLK + N_EXPERTS
NPAD = NB * GBLK

NC = 2
NS = 16
NW = NC * NS
APW = NA // NW


def _router_body(x_ref, choice_ref, kw_ref, pos_ref, meta_ref,
                 carry_ref, ind0_ref, ind1_ref, rk0_ref, rk1_ref):
    t = pl.program_id(0)
    x = x_ref[...]
    logits = lax.dot_general(
        x, choice_ref[...], (((1,), (1,)), ((), ())),
        preferred_element_type=jnp.float32,
    )
    eiota = lax.broadcasted_iota(jnp.int32, logits.shape, 1)
    valid = eiota < N_EXPERTS
    logits = jnp.where(valid, logits, -jnp.inf)
    m = jnp.max(logits, axis=1, keepdims=True)
    p = jnp.exp(logits - m)
    probs = p / jnp.sum(p, axis=1, keepdims=True)

    m1 = jnp.max(probs, axis=1, keepdims=True)
    i1 = jnp.min(jnp.where(probs == m1, eiota, EPAD), axis=1, keepdims=True)
    mask1 = eiota == i1
    probs2 = jnp.where(mask1 | ~valid, -1.0, probs)
    m2 = jnp.max(probs2, axis=1, keepdims=True)
    i2 = jnp.min(jnp.where(probs2 == m2, eiota, EPAD), axis=1, keepdims=True)
    mask2 = eiota == i2

    kw_ref[...] = jnp.concatenate([m1, m2], axis=1)

    ind0 = mask1.astype(jnp.float32)
    ind1 = mask2.astype(jnp.float32)
    ind01 = ind0 + ind1
    ri = lax.broadcasted_iota(jnp.int32, (TBLK, TBLK), 0)
    ci = lax.broadcasted_iota(jnp.int32, (TBLK, TBLK), 1)
    lmat = (ri > ci).astype(jnp.float32)

    @pl.when(t == 0)
    def _():
        carry_ref[...] = jnp.zeros_like(carry_ref)

    carry = carry_ref[...]
    csum = carry + lax.dot_general(
        lmat, ind01, (((1,), (0,)), ((), ())),
        preferred_element_type=jnp.float32,
    )
    rank0 = jnp.sum(csum * ind0, axis=1, keepdims=True)
    rank1 = jnp.sum(csum * ind1, axis=1, keepdims=True)
    carry_ref[...] = carry + jnp.sum(ind01, axis=0, keepdims=True)

    sl = pl.ds(t * TBLK, TBLK)
    ind0_ref[sl, :] = ind0
    ind1_ref[sl, :] = ind1
    rk0_ref[sl, :] = lax.broadcast_in_dim(rank0, (TBLK, EPAD), (0, 1))
    rk1_ref[sl, :] = lax.broadcast_in_dim(rank1, (TBLK, EPAD), (0, 1))

    @pl.when(t == pl.num_programs(0) - 1)
    def _():
        cnt = carry_ref[...]
        blocks = jnp.floor(cnt * (1.0 / GBLK) + (GBLK - 1.0) / GBLK)
        padded = blocks * GBLK
        li = lax.broadcasted_iota(jnp.int32, (EPAD, EPAD), 0)
        lj = lax.broadcasted_iota(jnp.int32, (EPAD, EPAD), 1)
        excl = (li < lj).astype(jnp.float32)
        incl = (li <= lj).astype(jnp.float32)
        poff = lax.dot_general(
            padded, excl, (((1,), (0,)), ((), ())),
            preferred_element_type=jnp.float32,
        )
        meta = lax.dot_general(
            blocks, incl, (((1,), (0,)), ((), ())),
            preferred_element_type=jnp.float32,
        )
        meta_ref[...] = meta.astype(jnp.int32)
        base0 = jnp.sum(ind0_ref[...] * poff, axis=1, keepdims=True)
        base1 = jnp.sum(ind1_ref[...] * poff, axis=1, keepdims=True)
        pos0 = base0 + rk0_ref[:, 0:1]
        pos1 = base1 + rk1_ref[:, 0:1]
        pos_ref[...] = jnp.concatenate([pos0, pos1], axis=1).astype(jnp.int32)


def _router(x2, choice):
    choice_p = jnp.zeros((EPAD, D_MODEL), jnp.float32).at[:N_EXPERTS].set(choice)
    return pl.pallas_call(
        _router_body,
        grid=(S // TBLK,),
        in_specs=[
            pl.BlockSpec((TBLK, D_MODEL), lambda t: (t, 0)),
            pl.BlockSpec((EPAD, D_MODEL), lambda t: (0, 0)),
        ],
        out_specs=[
            pl.BlockSpec((TBLK, TOP_K), lambda t: (t, 0)),
            pl.BlockSpec((S, TOP_K), lambda t: (0, 0)),
            pl.BlockSpec((1, EPAD), lambda t: (0, 0)),
        ],
        out_shape=[
            jax.ShapeDtypeStruct((S, TOP_K), jnp.float32),
            jax.ShapeDtypeStruct((S, TOP_K), jnp.int32),
            jax.ShapeDtypeStruct((1, EPAD), jnp.int32),
        ],
        scratch_shapes=[
            pltpu.VMEM((1, EPAD), jnp.float32),
            pltpu.VMEM((S, EPAD), jnp.float32),
            pltpu.VMEM((S, EPAD), jnp.float32),
            pltpu.VMEM((S, EPAD), jnp.float32),
            pltpu.VMEM((S, EPAD), jnp.float32),
        ],
        compiler_params=pltpu.CompilerParams(
            dimension_semantics=("arbitrary",),
        ),
    )(x2, choice_p)


def _dispatch_body(pos_ref, x_ref, xs_ref, ibuf, posbuf, xbuf, sem0, sem1):
    wid = lax.axis_index("s") * NC + lax.axis_index("c")
    base_a = wid * APW

    lanes = lax.iota(jnp.int32, 16)
    for c in range(APW // 16):
        av = lanes + (base_a + c * 16)
        ibuf[pl.ds(c * 16, 16)] = lax.shift_right_logical(av, 1)
    pltpu.sync_copy(pos_ref.at[pl.ds(base_a, APW)], posbuf)
    g = pltpu.async_copy(x_ref.at[ibuf], xbuf, sem0)
    g.wait()
    sc = pltpu.async_copy(xbuf, xs_ref.at[posbuf], sem1)
    sc.wait()


def _dispatch(pos_flat, x2):
    mesh = plsc.VectorSubcoreMesh(core_axis_name="c", subcore_axis_name="s")
    k = pl.kernel(
        _dispatch_body,
        out_type=jax.ShapeDtypeStruct((NPAD, D_MODEL), jnp.float32),
        mesh=mesh,
        scratch_types=[
            pltpu.VMEM((APW,), jnp.int32),
            pltpu.VMEM((APW,), jnp.int32),
            pltpu.VMEM((APW, D_MODEL), jnp.float32),
            pltpu.SemaphoreType.DMA,
            pltpu.SemaphoreType.DMA,
        ],
    )
    return k(pos_flat, x2)


def _gmm_body(meta_ref, xs_ref, w1a_ref, w1b_ref, w2a_ref, w2b_ref, ys_ref):
    b = pl.program_id(0)

    @pl.when(b < meta_ref[N_EXPERTS - 1])
    def _():
        xb = xs_ref[...]
        h1 = lax.dot_general(
            xb, w1a_ref[0], (((1,), (1,)), ((), ())),
            preferred_element_type=jnp.float32,
        )
        h1 = h1 * jax.nn.sigmoid(h1)
        h2 = lax.dot_general(
            xb, w1b_ref[0], (((1,), (1,)), ((), ())),
            preferred_element_type=jnp.float32,
        )
        h2 = h2 * jax.nn.sigmoid(h2)
        y1 = lax.dot_general(
            h1, w2a_ref[0], (((1,), (1,)), ((), ())),
            preferred_element_type=jnp.float32,
        )
        y2 = lax.dot_general(
            h2, w2b_ref[0], (((1,), (1,)), ((), ())),
            preferred_element_type=jnp.float32,
        )
        ys_ref[...] = y1 + y2


def _block_expert(b, meta_ref):
    bc = jnp.minimum(b, meta_ref[N_EXPERTS - 1] - 1)
    e = jnp.zeros((), jnp.int32)
    for i in range(N_EXPERTS):
        e += (bc >= meta_ref[i]).astype(jnp.int32)
    return e


def _gmm(meta_flat, xs, w1, w2):
    grid_spec = pltpu.PrefetchScalarGridSpec(
        num_scalar_prefetch=1,
        grid=(NB,),
        in_specs=[
            pl.BlockSpec(
                (GBLK, D_MODEL),
                lambda b, m: (jnp.minimum(b, m[N_EXPERTS - 1] - 1), 0)),
            pl.BlockSpec(
                (1, D_FFN // 2, D_MODEL),
                lambda b, m: (_block_expert(b, m), 0, 0)),
            pl.BlockSpec(
                (1, D_FFN // 2, D_MODEL),
                lambda b, m: (_block_expert(b, m), 1, 0)),
            pl.BlockSpec(
                (1, D_MODEL, D_FFN // 2),
                lambda b, m: (_block_expert(b, m), 0, 0)),
            pl.BlockSpec(
                (1, D_MODEL, D_FFN // 2),
                lambda b, m: (_block_expert(b, m), 0, 1)),
        ],
        out_specs=pl.BlockSpec((GBLK, D_MODEL), lambda b, m: (b, 0)),
    )
    return pl.pallas_call(
        _gmm_body,
        grid_spec=grid_spec,
        out_shape=jax.ShapeDtypeStruct((NPAD, D_MODEL), jnp.float32),
        compiler_params=pltpu.CompilerParams(
            dimension_semantics=("arbitrary",),
        ),
    )(meta_flat, xs, w1, w1, w2, w2)


TPW = S // NW


def _bcast16(v, j):
    idx = jnp.full((16,), j, jnp.int32)
    return lax.gather(
        v, idx[:, None],
        lax.GatherDimensionNumbers(
            offset_dims=(), collapsed_slice_dims=(0,), start_index_map=(0,)),
        slice_sizes=(1,),
        mode=lax.GatherScatterMode.PROMISE_IN_BOUNDS)


def _gather16(v, idx):
    return lax.gather(
        v, idx[:, None],
        lax.GatherDimensionNumbers(
            offset_dims=(), collapsed_slice_dims=(0,), start_index_map=(0,)),
        slice_sizes=(1,),
        mode=lax.GatherScatterMode.PROMISE_IN_BOUNDS)


def _combine_body(pos_ref, kw_ref, ys_ref, out_ref,
                  posbuf, kwbuf, idx0buf, idx1buf, rows0, rows1, sem0, sem1):
    wid = lax.axis_index("s") * NC + lax.axis_index("c")
    base_a = wid * APW
    base_t = wid * TPW

    pltpu.sync_copy(pos_ref.at[pl.ds(base_a, APW)], posbuf)
    pltpu.sync_copy(kw_ref.at[pl.ds(base_a, APW)], kwbuf)

    lanes = lax.iota(jnp.int32, 16)
    half = lanes < 8
    ev_lo = jnp.minimum(lanes * 2, 15)
    ev_hi = jnp.maximum(lanes * 2 - 16, 0)
    od_lo = jnp.minimum(lanes * 2 + 1, 15)
    od_hi = jnp.clip(lanes * 2 - 15, 0, 15)
    for c in range(TPW // 16):
        a_lo = posbuf[pl.ds(c * 32, 16)]
        a_hi = posbuf[pl.ds(c * 32 + 16, 16)]
        idx0buf[pl.ds(c * 16, 16)] = jnp.where(
            half, _gather16(a_lo, ev_lo), _gather16(a_hi, ev_hi))
        idx1buf[pl.ds(c * 16, 16)] = jnp.where(
            half, _gather16(a_lo, od_lo), _gather16(a_hi, od_hi))

    g0 = pltpu.async_copy(ys_ref.at[idx0buf], rows0, sem0)
    g1 = pltpu.async_copy(ys_ref.at[idx1buf], rows1, sem1)
    g0.wait()
    g1.wait()

    for g in range(TPW // 8):
        wv = kwbuf[pl.ds(16 * g, 16)]

        def body(j, _, wv=wv, g=g):
            t = g * 8 + j
            w0 = _bcast16(wv, 2 * j)
            w1 = _bcast16(wv, 2 * j + 1)
            for d in range(D_MODEL // 16):
                sl = pl.ds(d * 16, 16)
                rows0[t, sl] = rows0[t, sl] * w0 + rows1[t, sl] * w1
            return 0

        lax.fori_loop(0, 8, body, 0)
    pltpu.sync_copy(rows0, out_ref.at[pl.ds(base_t, TPW), :])


def _combine(pos_flat, kw_flat, ys):
    mesh = plsc.VectorSubcoreMesh(core_axis_name="c", subcore_axis_name="s")
    k = pl.kernel(
        _combine_body,
        out_type=jax.ShapeDtypeStruct((S, D_MODEL), jnp.float32),
        mesh=mesh,
        scratch_types=[
            pltpu.VMEM((APW,), jnp.int32),
            pltpu.VMEM((APW,), jnp.float32),
            pltpu.VMEM((TPW,), jnp.int32),
            pltpu.VMEM((TPW,), jnp.int32),
            pltpu.VMEM((TPW, D_MODEL), jnp.float32),
            pltpu.VMEM((TPW, D_MODEL), jnp.float32),
            pltpu.SemaphoreType.DMA,
            pltpu.SemaphoreType.DMA,
        ],
    )
    return k(pos_flat, kw_flat, ys)


@jax.jit
def kernel(x, choice, w1, w2):
    b, s, d = x.shape
    x2 = x.reshape(s, d)
    kw, pos, meta = _router(x2, choice)
    pos_flat = pos.reshape(-1)
    xs = _dispatch(pos_flat, x2)
    ys = _gmm(meta.reshape(-1), xs, w1, w2)
    out = _combine(pos_flat, kw.reshape(-1), ys)
    return out.reshape(b, s, d)


# speedup vs baseline: 1.0265x; 1.0265x over previous
import functools

import jax
import jax.numpy as jnp
from jax import lax
from jax.experimental import pallas as pl
from jax.experimental.pallas import tpu as pltpu
from jax.experimental.pallas import tpu_sc as plsc

S = 2048
D_MODEL = 768
D_FFN = 1536
N_EXPERTS = 8
TOP_K = 2

TBLK = 256
EPAD = 128
GBLK = 512
NA = S * TOP_K
NB = NA // GBLK + N_EXPERTS
NPAD = NB * GBLK

NC = 2
NS = 16
NW = NC * NS
APW = NA // NW


def _router_body(x_ref, choice_ref, kw_ref, pos_ref, meta_ref,
                 carry_ref, ind0_ref, ind1_ref, rk0_ref, rk1_ref):
    t = pl.program_id(0)
    E = N_EXPERTS
    x = x_ref[...]
    logits = lax.dot_general(
        x, choice_ref[...], (((1,), (1,)), ((), ())),
        preferred_element_type=jnp.float32,
    )
    eiota = lax.broadcasted_iota(jnp.int32, logits.shape, 1)
    m = jnp.max(logits, axis=1, keepdims=True)
    p = jnp.exp(logits - m)
    probs = p / jnp.sum(p, axis=1, keepdims=True)

    m1 = jnp.max(probs, axis=1, keepdims=True)
    i1 = jnp.min(jnp.where(probs == m1, eiota, E), axis=1, keepdims=True)
    mask1 = eiota == i1
    probs2 = jnp.where(mask1, -1.0, probs)
    m2 = jnp.max(probs2, axis=1, keepdims=True)
    i2 = jnp.min(jnp.where(probs2 == m2, eiota, E), axis=1, keepdims=True)
    mask2 = eiota == i2

    kw_ref[...] = jnp.concatenate([m1, m2], axis=1)

    ind0 = mask1.astype(jnp.float32)
    ind1 = mask2.astype(jnp.float32)
    ind01 = ind0 + ind1
    ri = lax.broadcasted_iota(jnp.int32, (TBLK, TBLK), 0)
    ci = lax.broadcasted_iota(jnp.int32, (TBLK, TBLK), 1)
    lmat = (ri > ci).astype(jnp.float32)

    @pl.when(t == 0)
    def _():
        carry_ref[...] = jnp.zeros_like(carry_ref)

    carry = carry_ref[...]
    csum = carry + lax.dot_general(
        lmat, ind01, (((1,), (0,)), ((), ())),
        preferred_element_type=jnp.float32,
    )
    rank0 = jnp.sum(csum * ind0, axis=1, keepdims=True)
    rank1 = jnp.sum(csum * ind1, axis=1, keepdims=True)
    carry_ref[...] = carry + jnp.sum(ind01, axis=0, keepdims=True)

    sl = pl.ds(t * TBLK, TBLK)
    ind0_ref[sl, :] = ind0
    ind1_ref[sl, :] = ind1
    rk0_ref[sl, :] = lax.broadcast_in_dim(rank0, (TBLK, E), (0, 1))
    rk1_ref[sl, :] = lax.broadcast_in_dim(rank1, (TBLK, E), (0, 1))

    @pl.when(t == pl.num_programs(0) - 1)
    def _():
        cnt = carry_ref[...]
        blocks = jnp.floor(cnt * (1.0 / GBLK) + (GBLK - 1.0) / GBLK)
        padded = blocks * GBLK
        li = lax.broadcasted_iota(jnp.int32, (E, E), 0)
        lj = lax.broadcasted_iota(jnp.int32, (E, E), 1)
        excl = (li < lj).astype(jnp.float32)
        incl = (li <= lj).astype(jnp.float32)
        poff = lax.dot_general(
            padded, excl, (((1,), (0,)), ((), ())),
            preferred_element_type=jnp.float32,
        )
        meta = lax.dot_general(
            blocks, incl, (((1,), (0,)), ((), ())),
            preferred_element_type=jnp.float32,
        )
        meta_ref[...] = meta.astype(jnp.int32)
        base0 = jnp.sum(ind0_ref[...] * poff, axis=1, keepdims=True)
        base1 = jnp.sum(ind1_ref[...] * poff, axis=1, keepdims=True)
        pos0 = base0 + rk0_ref[:, 0:1]
        pos1 = base1 + rk1_ref[:, 0:1]
        pos_ref[...] = jnp.concatenate([pos0, pos1], axis=1).astype(jnp.int32)


def _router(x2, choice):
    E = N_EXPERTS
    return pl.pallas_call(
        _router_body,
        grid=(S // TBLK,),
        in_specs=[
            pl.BlockSpec((TBLK, D_MODEL), lambda t: (t, 0)),
            pl.BlockSpec((E, D_MODEL), lambda t: (0, 0)),
        ],
        out_specs=[
            pl.BlockSpec((TBLK, TOP_K), lambda t: (t, 0)),
            pl.BlockSpec((S, TOP_K), lambda t: (0, 0)),
            pl.BlockSpec((1, E), lambda t: (0, 0)),
        ],
        out_shape=[
            jax.ShapeDtypeStruct((S, TOP_K), jnp.float32),
            jax.ShapeDtypeStruct((S, TOP_K), jnp.int32),
            jax.ShapeDtypeStruct((1, E), jnp.int32),
        ],
        scratch_shapes=[
            pltpu.VMEM((1, E), jnp.float32),
            pltpu.VMEM((S, E), jnp.float32),
            pltpu.VMEM((S, E), jnp.float32),
            pltpu.VMEM((S, E), jnp.float32),
            pltpu.VMEM((S, E), jnp.float32),
        ],
        compiler_params=pltpu.CompilerParams(
            dimension_semantics=("arbitrary",),
        ),
    )(x2, choice)


def _dispatch_body(pos_ref, x_ref, xs_ref, ibuf, posbuf, xbuf, sem0, sem1):
    wid = lax.axis_index("s") * NC + lax.axis_index("c")
    base_a = wid * APW

    lanes = lax.iota(jnp.int32, 16)
    for c in range(APW // 16):
        av = lanes + (base_a + c * 16)
        ibuf[pl.ds(c * 16, 16)] = lax.shift_right_logical(av, 1)
    pltpu.sync_copy(pos_ref.at[pl.ds(base_a, APW)], posbuf)
    g = pltpu.async_copy(x_ref.at[ibuf], xbuf, sem0)
    g.wait()
    sc = pltpu.async_copy(xbuf, xs_ref.at[posbuf], sem1)
    sc.wait()


def _dispatch(pos_flat, x2):
    mesh = plsc.VectorSubcoreMesh(core_axis_name="c", subcore_axis_name="s")
    k = pl.kernel(
        _dispatch_body,
        out_type=jax.ShapeDtypeStruct((NPAD, D_MODEL), jnp.float32),
        mesh=mesh,
        scratch_types=[
            pltpu.VMEM((APW,), jnp.int32),
            pltpu.VMEM((APW,), jnp.int32),
            pltpu.VMEM((APW, D_MODEL), jnp.float32),
            pltpu.SemaphoreType.DMA,
            pltpu.SemaphoreType.DMA,
        ],
    )
    return k(pos_flat, x2)


def _gmm_body(meta_ref, xs_ref, w1_ref, w2_ref, ys_ref):
    b = pl.program_id(0)

    @pl.when(b < meta_ref[0, N_EXPERTS - 1])
    def _():
        xb = xs_ref[...]
        h = lax.dot_general(
            xb, w1_ref[0], (((1,), (1,)), ((), ())),
            preferred_element_type=jnp.float32,
        )
        h = h * jax.nn.sigmoid(h)
        ys_ref[...] = lax.dot_general(
            h, w2_ref[0], (((1,), (1,)), ((), ())),
            preferred_element_type=jnp.float32,
        )


def _block_expert(b, meta_ref):
    bc = jnp.minimum(b, meta_ref[0, N_EXPERTS - 1] - 1)
    e = jnp.zeros((), jnp.int32)
    for i in range(N_EXPERTS):
        e += (bc >= meta_ref[0, i]).astype(jnp.int32)
    return e


def _gmm(meta, xs, w1, w2):
    grid_spec = pltpu.PrefetchScalarGridSpec(
        num_scalar_prefetch=1,
        grid=(NB,),
        in_specs=[
            pl.BlockSpec(
                (GBLK, D_MODEL),
                lambda b, m: (jnp.minimum(b, m[0, N_EXPERTS - 1] - 1), 0)),
            pl.BlockSpec(
                (1, D_FFN, D_MODEL),
                lambda b, m: (_block_expert(b, m), 0, 0)),
            pl.BlockSpec(
                (1, D_MODEL, D_FFN),
                lambda b, m: (_block_expert(b, m), 0, 0)),
        ],
        out_specs=pl.BlockSpec((GBLK, D_MODEL), lambda b, m: (b, 0)),
    )
    return pl.pallas_call(
        _gmm_body,
        grid_spec=grid_spec,
        out_shape=jax.ShapeDtypeStruct((NPAD, D_MODEL), jnp.float32),
        compiler_params=pltpu.CompilerParams(
            dimension_semantics=("arbitrary",),
        ),
    )(meta, xs, w1, w2)


TPW = S // NW


def _bcast16(v, j):
    idx = jnp.full((16,), j, jnp.int32)
    return lax.gather(
        v, idx[:, None],
        lax.GatherDimensionNumbers(
            offset_dims=(), collapsed_slice_dims=(0,), start_index_map=(0,)),
        slice_sizes=(1,),
        mode=lax.GatherScatterMode.PROMISE_IN_BOUNDS)


def _gather16(v, idx):
    return lax.gather(
        v, idx[:, None],
        lax.GatherDimensionNumbers(
            offset_dims=(), collapsed_slice_dims=(0,), start_index_map=(0,)),
        slice_sizes=(1,),
        mode=lax.GatherScatterMode.PROMISE_IN_BOUNDS)


def _combine_body(pos_ref, kw_ref, ys_ref, out_ref,
                  posbuf, kwbuf, idx0buf, idx1buf, rows0, rows1, sem0, sem1):
    wid = lax.axis_index("s") * NC + lax.axis_index("c")
    base_a = wid * APW
    base_t = wid * TPW

    pltpu.sync_copy(pos_ref.at[pl.ds(base_a, APW)], posbuf)
    pltpu.sync_copy(kw_ref.at[pl.ds(base_a, APW)], kwbuf)

    lanes = lax.iota(jnp.int32, 16)
    half = lanes < 8
    ev_lo = jnp.minimum(lanes * 2, 15)
    ev_hi = jnp.maximum(lanes * 2 - 16, 0)
    od_lo = jnp.minimum(lanes * 2 + 1, 15)
    od_hi = jnp.clip(lanes * 2 - 15, 0, 15)
    for c in range(TPW // 16):
        a_lo = posbuf[pl.ds(c * 32, 16)]
        a_hi = posbuf[pl.ds(c * 32 + 16, 16)]
        idx0buf[pl.ds(c * 16, 16)] = jnp.where(
            half, _gather16(a_lo, ev_lo), _gather16(a_hi, ev_hi))
        idx1buf[pl.ds(c * 16, 16)] = jnp.where(
            half, _gather16(a_lo, od_lo), _gather16(a_hi, od_hi))

    g0 = pltpu.async_copy(ys_ref.at[idx0buf], rows0, sem0)
    g1 = pltpu.async_copy(ys_ref.at[idx1buf], rows1, sem1)
    g0.wait()
    g1.wait()

    for g in range(TPW // 8):
        wv = kwbuf[pl.ds(16 * g, 16)]

        def body(j, _, wv=wv, g=g):
            t = g * 8 + j
            w0 = _bcast16(wv, 2 * j)
            w1 = _bcast16(wv, 2 * j + 1)
            for d in range(D_MODEL // 16):
                sl = pl.ds(d * 16, 16)
                rows0[t, sl] = rows0[t, sl] * w0 + rows1[t, sl] * w1
            return 0

        lax.fori_loop(0, 8, body, 0)
    pltpu.sync_copy(rows0, out_ref.at[pl.ds(base_t, TPW), :])


def _combine(pos_flat, kw_flat, ys):
    mesh = plsc.VectorSubcoreMesh(core_axis_name="c", subcore_axis_name="s")
    k = pl.kernel(
        _combine_body,
        out_type=jax.ShapeDtypeStruct((S, D_MODEL), jnp.float32),
        mesh=mesh,
        scratch_types=[
            pltpu.VMEM((APW,), jnp.int32),
            pltpu.VMEM((APW,), jnp.float32),
            pltpu.VMEM((TPW,), jnp.int32),
            pltpu.VMEM((TPW,), jnp.int32),
            pltpu.VMEM((TPW, D_MODEL), jnp.float32),
            pltpu.VMEM((TPW, D_MODEL), jnp.float32),
            pltpu.SemaphoreType.DMA,
            pltpu.SemaphoreType.DMA,
        ],
    )
    return k(pos_flat, kw_flat, ys)


@jax.jit
def kernel(x, choice, w1, w2):
    b, s, d = x.shape
    x2 = x.reshape(s, d)
    kw, pos, meta = _router(x2, choice)
    pos_flat = pos.reshape(-1)
    xs = _dispatch(pos_flat, x2)
    ys = _gmm(meta, xs, w1, w2)
    out = _combine(pos_flat, kw.reshape(-1), ys)
    return out.reshape(b, s, d)


# speedup vs baseline: 1.0406x; 1.0137x over previous
import functools

import jax
import jax.numpy as jnp
from jax import lax
from jax.experimental import pallas as pl
from jax.experimental.pallas import tpu as pltpu
from jax.experimental.pallas import tpu_sc as plsc

S = 2048
D_MODEL = 768
D_FFN = 1536
N_EXPERTS = 8
TOP_K = 2

TBLK = 256
EPAD = 128
GBLK = 512
NA = S * TOP_K
NB = NA // GBLK + N_EXPERTS
NPAD = NB * GBLK

NC = 2
NS = 16
NW = NC * NS
APW = NA // NW


def _router_body(x_ref, choice_ref, kw_ref, pos_ref, meta_ref,
                 carry_ref, ind0_ref, ind1_ref, rk0_ref, rk1_ref):
    t = pl.program_id(0)
    E = N_EXPERTS
    x = x_ref[...]
    logits = lax.dot_general(
        x, choice_ref[...], (((1,), (1,)), ((), ())),
        preferred_element_type=jnp.float32,
    )
    eiota = lax.broadcasted_iota(jnp.int32, logits.shape, 1)
    m = jnp.max(logits, axis=1, keepdims=True)
    p = jnp.exp(logits - m)
    probs = p / jnp.sum(p, axis=1, keepdims=True)

    m1 = jnp.max(probs, axis=1, keepdims=True)
    i1 = jnp.min(jnp.where(probs == m1, eiota, E), axis=1, keepdims=True)
    mask1 = eiota == i1
    probs2 = jnp.where(mask1, -1.0, probs)
    m2 = jnp.max(probs2, axis=1, keepdims=True)
    i2 = jnp.min(jnp.where(probs2 == m2, eiota, E), axis=1, keepdims=True)
    mask2 = eiota == i2

    kw_ref[...] = jnp.concatenate([m1, m2], axis=1)

    ind0 = mask1.astype(jnp.float32)
    ind1 = mask2.astype(jnp.float32)
    ind01 = ind0 + ind1
    ri = lax.broadcasted_iota(jnp.int32, (TBLK, TBLK), 0)
    ci = lax.broadcasted_iota(jnp.int32, (TBLK, TBLK), 1)
    lmat = (ri > ci).astype(jnp.float32)

    @pl.when(t == 0)
    def _():
        carry_ref[...] = jnp.zeros_like(carry_ref)

    carry = carry_ref[...]
    csum = carry + lax.dot_general(
        lmat, ind01, (((1,), (0,)), ((), ())),
        preferred_element_type=jnp.float32,
    )
    rank0 = jnp.sum(csum * ind0, axis=1, keepdims=True)
    rank1 = jnp.sum(csum * ind1, axis=1, keepdims=True)
    carry_ref[...] = carry + jnp.sum(ind01, axis=0, keepdims=True)

    sl = pl.ds(t * TBLK, TBLK)
    ind0_ref[sl, :] = ind0
    ind1_ref[sl, :] = ind1
    rk0_ref[sl, :] = lax.broadcast_in_dim(rank0, (TBLK, E), (0, 1))
    rk1_ref[sl, :] = lax.broadcast_in_dim(rank1, (TBLK, E), (0, 1))

    @pl.when(t == pl.num_programs(0) - 1)
    def _():
        cnt = carry_ref[...]
        blocks = jnp.floor(cnt * (1.0 / GBLK) + (GBLK - 1.0) / GBLK)
        padded = blocks * GBLK
        li = lax.broadcasted_iota(jnp.int32, (E, E), 0)
        lj = lax.broadcasted_iota(jnp.int32, (E, E), 1)
        excl = (li < lj).astype(jnp.float32)
        incl = (li <= lj).astype(jnp.float32)
        poff = lax.dot_general(
            padded, excl, (((1,), (0,)), ((), ())),
            preferred_element_type=jnp.float32,
        )
        meta = lax.dot_general(
            blocks, incl, (((1,), (0,)), ((), ())),
            preferred_element_type=jnp.float32,
        )
        meta_ref[...] = meta.astype(jnp.int32)
        base0 = jnp.sum(ind0_ref[...] * poff, axis=1, keepdims=True)
        base1 = jnp.sum(ind1_ref[...] * poff, axis=1, keepdims=True)
        pos0 = base0 + rk0_ref[:, 0:1]
        pos1 = base1 + rk1_ref[:, 0:1]
        pos_ref[...] = jnp.concatenate([pos0, pos1], axis=1).astype(jnp.int32)


def _router(x2, choice):
    E = N_EXPERTS
    return pl.pallas_call(
        _router_body,
        grid=(S // TBLK,),
        in_specs=[
            pl.BlockSpec((TBLK, D_MODEL), lambda t: (t, 0)),
            pl.BlockSpec((E, D_MODEL), lambda t: (0, 0)),
        ],
        out_specs=[
            pl.BlockSpec((TBLK, TOP_K), lambda t: (t, 0)),
            pl.BlockSpec((S, TOP_K), lambda t: (0, 0)),
            pl.BlockSpec((1, E), lambda t: (0, 0)),
        ],
        out_shape=[
            jax.ShapeDtypeStruct((S, TOP_K), jnp.float32),
            jax.ShapeDtypeStruct((S, TOP_K), jnp.int32),
            jax.ShapeDtypeStruct((1, E), jnp.int32),
        ],
        scratch_shapes=[
            pltpu.VMEM((1, E), jnp.float32),
            pltpu.VMEM((S, E), jnp.float32),
            pltpu.VMEM((S, E), jnp.float32),
            pltpu.VMEM((S, E), jnp.float32),
            pltpu.VMEM((S, E), jnp.float32),
        ],
        compiler_params=pltpu.CompilerParams(
            dimension_semantics=("arbitrary",),
        ),
    )(x2, choice)


def _dispatch_body(pos_ref, x_ref, xs_ref, ibuf, posbuf, xbuf, sem0, sem1):
    wid = lax.axis_index("s") * NC + lax.axis_index("c")
    base_a = wid * APW

    lanes = lax.iota(jnp.int32, 16)
    for c in range(APW // 16):
        av = lanes + (base_a + c * 16)
        ibuf[pl.ds(c * 16, 16)] = lax.shift_right_logical(av, 1)
    pltpu.sync_copy(pos_ref.at[pl.ds(base_a, APW)], posbuf)
    g = pltpu.async_copy(x_ref.at[ibuf], xbuf, sem0)
    g.wait()
    sc = pltpu.async_copy(xbuf, xs_ref.at[posbuf], sem1)
    sc.wait()


def _dispatch(pos_flat, x2):
    mesh = plsc.VectorSubcoreMesh(core_axis_name="c", subcore_axis_name="s")
    k = pl.kernel(
        _dispatch_body,
        out_type=jax.ShapeDtypeStruct((NPAD, D_MODEL), jnp.float32),
        mesh=mesh,
        scratch_types=[
            pltpu.VMEM((APW,), jnp.int32),
            pltpu.VMEM((APW,), jnp.int32),
            pltpu.VMEM((APW, D_MODEL), jnp.float32),
            pltpu.SemaphoreType.DMA,
            pltpu.SemaphoreType.DMA,
        ],
    )
    return k(pos_flat, x2)


def _gmm_body(meta_ref, xs_ref, w1_ref, w2_ref, ys_ref):
    b = pl.program_id(0)

    @pl.when(b < meta_ref[0, N_EXPERTS - 1])
    def _():
        xb = xs_ref[...]
        h = lax.dot_general(
            xb, w1_ref[0], (((1,), (1,)), ((), ())),
            preferred_element_type=jnp.float32,
        )
        h = h * jax.nn.sigmoid(h)
        ys_ref[...] = lax.dot_general(
            h, w2_ref[0], (((1,), (1,)), ((), ())),
            preferred_element_type=jnp.float32,
        )


def _block_expert(b, meta_ref):
    bc = jnp.minimum(b, meta_ref[0, N_EXPERTS - 1] - 1)
    e = jnp.zeros((), jnp.int32)
    for i in range(N_EXPERTS):
        e += (bc >= meta_ref[0, i]).astype(jnp.int32)
    return e


def _gmm(meta, xs, w1, w2):
    grid_spec = pltpu.PrefetchScalarGridSpec(
        num_scalar_prefetch=1,
        grid=(NB,),
        in_specs=[
            pl.BlockSpec(
                (GBLK, D_MODEL),
                lambda b, m: (jnp.minimum(b, m[0, N_EXPERTS - 1] - 1), 0)),
            pl.BlockSpec(
                (1, D_FFN, D_MODEL),
                lambda b, m: (_block_expert(b, m), 0, 0)),
            pl.BlockSpec(
                (1, D_MODEL, D_FFN),
                lambda b, m: (_block_expert(b, m), 0, 0)),
        ],
        out_specs=pl.BlockSpec((GBLK, D_MODEL), lambda b, m: (b, 0)),
    )
    return pl.pallas_call(
        _gmm_body,
        grid_spec=grid_spec,
        out_shape=jax.ShapeDtypeStruct((NPAD, D_MODEL), jnp.float32),
        compiler_params=pltpu.CompilerParams(
            dimension_semantics=("arbitrary",),
        ),
    )(meta, xs, w1, w2)


TPW = S // NW


def _bcast16(v, j):
    idx = jnp.full((16,), j, jnp.int32)
    return lax.gather(
        v, idx[:, None],
        lax.GatherDimensionNumbers(
            offset_dims=(), collapsed_slice_dims=(0,), start_index_map=(0,)),
        slice_sizes=(1,),
        mode=lax.GatherScatterMode.PROMISE_IN_BOUNDS)


def _gather16(v, idx):
    return lax.gather(
        v, idx[:, None],
        lax.GatherDimensionNumbers(
            offset_dims=(), collapsed_slice_dims=(0,), start_index_map=(0,)),
        slice_sizes=(1,),
        mode=lax.GatherScatterMode.PROMISE_IN_BOUNDS)


def _combine_body(pos_ref, kw_ref, ys_ref, out_ref,
                  posbuf, kwbuf, idx0buf, idx1buf, rows0, rows1, sem0, sem1):
    wid = lax.axis_index("s") * NC + lax.axis_index("c")
    base_a = wid * APW
    base_t = wid * TPW

    pltpu.sync_copy(pos_ref.at[pl.ds(base_a, APW)], posbuf)
    pltpu.sync_copy(kw_ref.at[pl.ds(base_a, APW)], kwbuf)

    lanes = lax.iota(jnp.int32, 16)
    half = lanes < 8
    ev_lo = jnp.minimum(lanes * 2, 15)
    ev_hi = jnp.maximum(lanes * 2 - 16, 0)
    od_lo = jnp.minimum(lanes * 2 + 1, 15)
    od_hi = jnp.clip(lanes * 2 - 15, 0, 15)
    for c in range(TPW // 16):
        a_lo = posbuf[pl.ds(c * 32, 16)]
        a_hi = posbuf[pl.ds(c * 32 + 16, 16)]
        idx0buf[pl.ds(c * 16, 16)] = jnp.where(
            half, _gather16(a_lo, ev_lo), _gather16(a_hi, ev_hi))
        idx1buf[pl.ds(c * 16, 16)] = jnp.where(
            half, _gather16(a_lo, od_lo), _gather16(a_hi, od_hi))

    HPW = TPW // 2
    halves = (
        (rows0.at[pl.ds(0, HPW), :], rows1.at[pl.ds(0, HPW), :], sem0, 0),
        (rows0.at[pl.ds(HPW, HPW), :], rows1.at[pl.ds(HPW, HPW), :], sem1, 1),
    )
    cps = []
    for (r0h, r1h, sem, hh) in halves:
        cps.append((
            pltpu.async_copy(ys_ref.at[idx0buf.at[pl.ds(hh * HPW, HPW)]],
                             r0h, sem),
            pltpu.async_copy(ys_ref.at[idx1buf.at[pl.ds(hh * HPW, HPW)]],
                             r1h, sem),
        ))

    out_cps = []
    for (r0h, r1h, sem, hh) in halves:
        cps[hh][0].wait()
        cps[hh][1].wait()
        for g in range(HPW // 8):
            wv = kwbuf[pl.ds(hh * 2 * HPW + 16 * g, 16)]

            def body(j, _, wv=wv, g=g, hh=hh):
                t = hh * HPW + g * 8 + j
                w0 = _bcast16(wv, 2 * j)
                w1 = _bcast16(wv, 2 * j + 1)
                for d in range(D_MODEL // 16):
                    sl = pl.ds(d * 16, 16)
                    rows0[t, sl] = rows0[t, sl] * w0 + rows1[t, sl] * w1
                return 0

            lax.fori_loop(0, 8, body, 0)
        out_cps.append(pltpu.async_copy(
            r0h, out_ref.at[pl.ds(base_t + hh * HPW, HPW), :], sem))
    for cp in out_cps:
        cp.wait()


def _combine(pos_flat, kw_flat, ys):
    mesh = plsc.VectorSubcoreMesh(core_axis_name="c", subcore_axis_name="s")
    k = pl.kernel(
        _combine_body,
        out_type=jax.ShapeDtypeStruct((S, D_MODEL), jnp.float32),
        mesh=mesh,
        scratch_types=[
            pltpu.VMEM((APW,), jnp.int32),
            pltpu.VMEM((APW,), jnp.float32),
            pltpu.VMEM((TPW,), jnp.int32),
            pltpu.VMEM((TPW,), jnp.int32),
            pltpu.VMEM((TPW, D_MODEL), jnp.float32),
            pltpu.VMEM((TPW, D_MODEL), jnp.float32),
            pltpu.SemaphoreType.DMA,
            pltpu.SemaphoreType.DMA,
        ],
    )
    return k(pos_flat, kw_flat, ys)


@jax.jit
def kernel(x, choice, w1, w2):
    b, s, d = x.shape
    x2 = x.reshape(s, d)
    kw, pos, meta = _router(x2, choice)
    pos_flat = pos.reshape(-1)
    xs = _dispatch(pos_flat, x2)
    ys = _gmm(meta, xs, w1, w2)
    out = _combine(pos_flat, kw.reshape(-1), ys)
    return out.reshape(b, s, d)
